# baseline (speedup 1.0000x reference)
import jax
import jax.numpy as jnp
from jax import lax
from jax.experimental import pallas as pl
from jax.experimental.pallas import tpu as pltpu

B = 128
V = 100000
RS = 8
NG = B // RS
NSH = 16
SH = 6272
BIG = 2**30


def _tc_shard_argmax(x_ref, v_ref, i_ref):
    for s in range(NSH):
        lo = s * SH
        hi = min(lo + SH, V)
        sl = x_ref[:, lo:hi]
        m = jnp.max(sl, axis=1, keepdims=True)
        iota = lax.broadcasted_iota(jnp.int32, (RS, hi - lo), 1)
        cand = jnp.where(sl == m, iota, BIG)
        idx = jnp.min(cand, axis=1, keepdims=True) + lo
        v_ref[:, s:s + 1] = m
        i_ref[:, s:s + 1] = idx


@jax.jit
def kernel(x):
    v2, i2 = pl.pallas_call(
        _tc_shard_argmax,
        grid=(NG,),
        in_specs=[pl.BlockSpec((RS, V), lambda g: (g, 0))],
        out_specs=[
            pl.BlockSpec((RS, NSH), lambda g: (g, 0)),
            pl.BlockSpec((RS, NSH), lambda g: (g, 0)),
        ],
        out_shape=[
            jax.ShapeDtypeStruct((B, NSH), jnp.float32),
            jax.ShapeDtypeStruct((B, NSH), jnp.int32),
        ],
    )(x)

    m = jnp.max(v2, axis=1, keepdims=True)
    idx = jnp.min(jnp.where(v2 == m, i2, BIG), axis=1)
    return idx.astype(jnp.int64)


# speedup vs baseline: 1.0049x; 1.0049x over previous
import jax
import jax.numpy as jnp
from jax import lax
from jax.experimental import pallas as pl
from jax.experimental.pallas import tpu as pltpu

B = 128
V = 100000
RS = 8
NG = B // RS
NSH = 16
SH = 6272
BIG = 2**30


def _shard_argmax_body(xb, v_ref, i_ref, half):
    r0 = half * RS
    for s in range(NSH):
        lo = s * SH
        hi = min(lo + SH, V)
        sl = xb[:, lo:hi]
        m = jnp.max(sl, axis=1, keepdims=True)
        iota = lax.broadcasted_iota(jnp.int32, (RS, hi - lo), 1)
        cand = jnp.where(sl == m, iota, BIG)
        idx = jnp.min(cand, axis=1, keepdims=True) + lo
        v_ref[r0:r0 + RS, s:s + 1] = m
        i_ref[r0:r0 + RS, s:s + 1] = idx


def _tc_shard_argmax(x_hbm, v_ref, i_ref, buf0, buf1, sem0, sem1):
    k = pl.program_id(0)
    bufs = (buf0, buf1)
    sems = (sem0, sem1)

    def copy(slab, slot):
        return pltpu.make_async_copy(
            x_hbm.at[pl.ds(slab * RS, RS), :], bufs[slot], sems[slot])

    @pl.when(k == 0)
    def _():
        copy(0, 0).start()

    copy(2 * k + 1, 1).start()
    copy(2 * k, 0).wait()
    _shard_argmax_body(buf0, v_ref, i_ref, 0)

    @pl.when(k + 1 < NG // 2)
    def _():
        copy(2 * k + 2, 0).start()

    copy(2 * k + 1, 1).wait()
    _shard_argmax_body(buf1, v_ref, i_ref, 1)


@jax.jit
def kernel(x):
    v2, i2 = pl.pallas_call(
        _tc_shard_argmax,
        grid=(NG // 2,),
        in_specs=[pl.BlockSpec(memory_space=pl.ANY)],
        out_specs=[
            pl.BlockSpec((2 * RS, NSH), lambda k: (k, 0)),
            pl.BlockSpec((2 * RS, NSH), lambda k: (k, 0)),
        ],
        out_shape=[
            jax.ShapeDtypeStruct((B, NSH), jnp.float32),
            jax.ShapeDtypeStruct((B, NSH), jnp.int32),
        ],
        scratch_shapes=[
            pltpu.VMEM((RS, V), jnp.float32),
            pltpu.VMEM((RS, V), jnp.float32),
            pltpu.SemaphoreType.DMA,
            pltpu.SemaphoreType.DMA,
        ],
    )(x)

    m = jnp.max(v2, axis=1, keepdims=True)
    idx = jnp.min(jnp.where(v2 == m, i2, BIG), axis=1)
    return idx.astype(jnp.int64)


# speedup vs baseline: 1.9961x; 1.9864x over previous
import jax
import jax.numpy as jnp
from jax import lax
from jax.experimental import pallas as pl

B = 128
V = 100000
BSV = 4000
NBV = V // BSV
BIG = 2**30


def _tc_shard_argmax(xt_ref, v_ref, i_ref):
    b = pl.program_id(0)
    xb = xt_ref[...]
    m = jnp.max(xb, axis=0, keepdims=True)
    iota = lax.broadcasted_iota(jnp.int32, (BSV, B), 0)
    cand = jnp.where(xb == m, iota, BIG)
    idx = jnp.min(cand, axis=0, keepdims=True) + b * BSV
    v_ref[...] = m.reshape(1, 1, B)
    i_ref[...] = idx.reshape(1, 1, B)


@jax.jit
def kernel(x):
    xt = jnp.swapaxes(x, 0, 1)
    v2, i2 = pl.pallas_call(
        _tc_shard_argmax,
        grid=(NBV,),
        in_specs=[pl.BlockSpec((BSV, B), lambda b: (b, 0))],
        out_specs=[
            pl.BlockSpec((1, 1, B), lambda b: (b, 0, 0)),
            pl.BlockSpec((1, 1, B), lambda b: (b, 0, 0)),
        ],
        out_shape=[
            jax.ShapeDtypeStruct((NBV, 1, B), jnp.float32),
            jax.ShapeDtypeStruct((NBV, 1, B), jnp.int32),
        ],
    )(xt)

    v2 = v2[:, 0, :]
    i2 = i2[:, 0, :]
    m = jnp.max(v2, axis=0, keepdims=True)
    idx = jnp.min(jnp.where(v2 == m, i2, BIG), axis=0)
    return idx.astype(jnp.int64)


# speedup vs baseline: 2.3723x; 1.1884x over previous
import jax
import jax.numpy as jnp
from jax import lax
from jax.experimental import pallas as pl

B = 128
V = 100000
BSV = 10000
NBV = V // BSV
BIG = 2**30


def _tc_shard_argmax(xt_ref, v_ref, i_ref):
    b = pl.program_id(0)
    xb = xt_ref[...]
    m = jnp.max(xb, axis=0, keepdims=True)
    iota = lax.broadcasted_iota(jnp.int32, (BSV, B), 0)
    cand = jnp.where(xb == m, iota, BIG)
    idx = jnp.min(cand, axis=0, keepdims=True) + b * BSV
    v_ref[...] = m.reshape(1, 1, B)
    i_ref[...] = idx.reshape(1, 1, B)


@jax.jit
def kernel(x):
    xt = jnp.swapaxes(x, 0, 1)
    v2, i2 = pl.pallas_call(
        _tc_shard_argmax,
        grid=(NBV,),
        in_specs=[pl.BlockSpec((BSV, B), lambda b: (b, 0))],
        out_specs=[
            pl.BlockSpec((1, 1, B), lambda b: (b, 0, 0)),
            pl.BlockSpec((1, 1, B), lambda b: (b, 0, 0)),
        ],
        out_shape=[
            jax.ShapeDtypeStruct((NBV, 1, B), jnp.float32),
            jax.ShapeDtypeStruct((NBV, 1, B), jnp.int32),
        ],
    )(xt)

    v2 = v2[:, 0, :]
    i2 = i2[:, 0, :]
    m = jnp.max(v2, axis=0, keepdims=True)
    idx = jnp.min(jnp.where(v2 == m, i2, BIG), axis=0)
    return idx.astype(jnp.int64)
